# baseline (speedup 1.0000x reference)
import functools
import jax
import jax.numpy as jnp
from jax import lax
from jax.experimental import pallas as pl
from jax.experimental.pallas import tpu as pltpu
from jax.experimental.pallas import tpu_sc as plsc

H = 256
G = 64
NUM_SC = 2
NUM_SUB = 16
EB = 256
D = 4096
DPAD = 16
RB = 512
SB = 3
ZR = 64


def _cdiv(a, b):
    return (a + b - 1) // b


NW = NUM_SC * NUM_SUB


@functools.lru_cache(maxsize=None)
def _make_seg_sum(e_pad, n_out):
    mesh = plsc.VectorSubcoreMesh(
        core_axis_name="c", subcore_axis_name="s",
        num_cores=NUM_SC, num_subcores=NUM_SUB)

    @functools.partial(
        pl.kernel,
        out_type=jax.ShapeDtypeStruct((n_out + DPAD, H), jnp.float32),
        mesh=mesh,
        scratch_types=[
            pltpu.VMEM((EB,), jnp.int32),
            pltpu.VMEM((EB,), jnp.int32),
            pltpu.VMEM((EB,), jnp.int32),
            pltpu.VMEM((EB, H), jnp.float32),
            pltpu.VMEM((ZR, H), jnp.float32),
            pltpu.VMEM((1, 16), jnp.int32),
            pltpu.SemaphoreType.DMA,
            pltpu.SemaphoreType.DMA,
        ],
    )
    def kern(x_hbm, srcs_hbm, dsts_hbm, pairs_hbm, zeros_hbm, out_hbm,
             sidx, didx, lidx, rows, ztile, pairref, gsem, ssem):
        cid = lax.axis_index("c")
        sid = lax.axis_index("s")
        wid = sid * NUM_SC + cid
        pltpu.sync_copy(zeros_hbm, ztile)
        pltpu.sync_copy(pairs_hbm.at[pl.ds(wid, 1)], pairref)
        p = pairref[0, pl.ds(0, 16)]
        e_lo, e_hi, r_lo, r_hi = p[0], p[1], p[2], p[3]

        def zero_body(i, _):
            r0 = pl.multiple_of(r_lo + i * ZR, ZR)
            pltpu.sync_copy(ztile, out_hbm.at[pl.ds(r0, ZR)])
            return 0
        lax.fori_loop(0, (r_hi - r_lo) // ZR, zero_body, 0)

        blo = e_lo // EB
        bhi = (e_hi + EB - 1) // EB

        def blk_body(i, _):
            e0 = pl.multiple_of((blo + i) * EB, EB)
            pltpu.sync_copy(srcs_hbm.at[pl.ds(e0, EB)], sidx)
            pltpu.sync_copy(dsts_hbm.at[pl.ds(e0, EB)], didx)
            for k in range(EB // 16):
                v = didx[pl.ds(k * 16, 16)]
                oob = (v < r_lo) | (v >= r_hi)
                lidx[pl.ds(k * 16, 16)] = jnp.where(oob, n_out, v)
            pltpu.async_copy(x_hbm.at[sidx], rows, gsem).wait()
            pltpu.async_copy(rows, out_hbm.at[lidx], ssem, add=True).wait()
            return 0
        lax.fori_loop(0, bhi - blo, blk_body, 0)

    return kern


def _embed_body(nt_ref, ct_ref, pt_ref, nte_ref, cte_ref, pte_ref, o_ref):
    blk = nt_ref.shape[0]
    def onehot(idx, w):
        i = lax.broadcasted_iota(jnp.int32, (blk, w), 1)
        return (idx[:, None] == i).astype(jnp.float32)
    o = jnp.dot(onehot(nt_ref[...], 8), nte_ref[...],
                preferred_element_type=jnp.float32)
    o += jnp.dot(onehot(ct_ref[...], 16), cte_ref[...],
                 preferred_element_type=jnp.float32)
    o += jnp.dot(onehot(pt_ref[...], 16), pte_ref[...],
                 preferred_element_type=jnp.float32)
    o_ref[...] = o


def _embed(nt, ct, pt, nte_p, cte_p, pte_p):
    n = nt.shape[0]
    grid = (n // RB,)
    return pl.pallas_call(
        _embed_body,
        grid=grid,
        in_specs=[
            pl.BlockSpec((RB,), lambda i: (i,)),
            pl.BlockSpec((RB,), lambda i: (i,)),
            pl.BlockSpec((RB,), lambda i: (i,)),
            pl.BlockSpec((8, H), lambda i: (0, 0)),
            pl.BlockSpec((16, H), lambda i: (0, 0)),
            pl.BlockSpec((16, H), lambda i: (0, 0)),
        ],
        out_specs=pl.BlockSpec((RB, H), lambda i: (i, 0)),
        out_shape=jax.ShapeDtypeStruct((n, H), jnp.float32),
    )(nt, ct, pt, nte_p, cte_p, pte_p)


def _matmul_body(x_ref, w_ref, o_ref):
    o_ref[...] = jnp.dot(x_ref[...], w_ref[...],
                         preferred_element_type=jnp.float32)


def _matmul(x, w):
    n = x.shape[0]
    return pl.pallas_call(
        _matmul_body,
        grid=(n // RB,),
        in_specs=[
            pl.BlockSpec((RB, H), lambda i: (i, 0)),
            pl.BlockSpec((H, H), lambda i: (0, 0)),
        ],
        out_specs=pl.BlockSpec((RB, H), lambda i: (i, 0)),
        out_shape=jax.ShapeDtypeStruct((n, H), jnp.float32),
    )(x, w)


def _combine1_body(s_ref, c_ref, x_ref, wl_ref, wr_ref, b_ref, o_ref):
    a = s_ref[...] / jnp.maximum(c_ref[...], 1.0)[:, None]
    o = jnp.dot(a, wl_ref[...], preferred_element_type=jnp.float32)
    o += jnp.dot(x_ref[...], wr_ref[...], preferred_element_type=jnp.float32)
    o_ref[...] = jnp.maximum(o + b_ref[...][None, :], 0.0)


def _combine1(s, c, x, wl, wr, b):
    n = x.shape[0]
    return pl.pallas_call(
        _combine1_body,
        grid=(n // RB,),
        in_specs=[
            pl.BlockSpec((RB, H), lambda i: (i, 0)),
            pl.BlockSpec((RB,), lambda i: (i,)),
            pl.BlockSpec((RB, H), lambda i: (i, 0)),
            pl.BlockSpec((H, H), lambda i: (0, 0)),
            pl.BlockSpec((H, H), lambda i: (0, 0)),
            pl.BlockSpec((H,), lambda i: (0,)),
        ],
        out_specs=pl.BlockSpec((RB, H), lambda i: (i, 0)),
        out_shape=jax.ShapeDtypeStruct((n, H), jnp.float32),
    )(s, c, x, wl, wr, b)


def _combine_pin_body(s1_ref, c1_ref, s2_ref, c2_ref, s3_ref, c3_ref,
                      x_ref, wl_ref, wr_ref, b_ref, o_ref):
    o = s1_ref[...] / jnp.maximum(c1_ref[...], 1.0)[:, None]
    o += s2_ref[...] / jnp.maximum(c2_ref[...], 1.0)[:, None]
    a3 = s3_ref[...] / jnp.maximum(c3_ref[...], 1.0)[:, None]
    o += jnp.dot(a3, wl_ref[...], preferred_element_type=jnp.float32)
    o += jnp.dot(x_ref[...], wr_ref[...], preferred_element_type=jnp.float32)
    o_ref[...] = jnp.maximum(o + b_ref[...][None, :], 0.0)


def _combine_pin(s1, c1, s2, c2, s3, c3, x, wl3, wr, b):
    n = x.shape[0]
    row = pl.BlockSpec((RB, H), lambda i: (i, 0))
    vec = pl.BlockSpec((RB,), lambda i: (i,))
    return pl.pallas_call(
        _combine_pin_body,
        grid=(n // RB,),
        in_specs=[row, vec, row, vec, row, vec, row,
                  pl.BlockSpec((H, H), lambda i: (0, 0)),
                  pl.BlockSpec((H, H), lambda i: (0, 0)),
                  pl.BlockSpec((H,), lambda i: (0,))],
        out_specs=row,
        out_shape=jax.ShapeDtypeStruct((n, H), jnp.float32),
    )(s1, c1, s2, c2, s3, c3, x, wl3, wr, b)


def _pool_body(comp_ref, batch_ref, sum_ref, cnt_ref, max_ref):
    i = pl.program_id(0)
    blk = comp_ref.shape[0]

    @pl.when(i == 0)
    def _init():
        sum_ref[...] = jnp.zeros_like(sum_ref)
        cnt_ref[...] = jnp.zeros_like(cnt_ref)
        max_ref[...] = jnp.full_like(max_ref, -3.0e38)

    comp = comp_ref[...]
    b = batch_ref[...]
    oh = (b[:, None] == lax.broadcasted_iota(jnp.int32, (blk, G), 1))
    ohf = oh.astype(jnp.float32)
    sum_ref[...] += jnp.dot(ohf.T, comp, preferred_element_type=jnp.float32)
    cnt_ref[...] += jnp.sum(ohf, axis=0)
    g0 = b[0]
    g1 = b[blk - 1]
    for g in range(G):
        @pl.when((g0 <= g) & (g <= g1))
        def _gmax():
            m = jnp.where(oh[:, g][:, None], comp, -3.0e38)
            max_ref[g, :] = jnp.maximum(max_ref[g, :], jnp.max(m, axis=0))


def _pool(comp_pad, batch_pad):
    n = comp_pad.shape[0]
    return pl.pallas_call(
        _pool_body,
        grid=(n // RB,),
        in_specs=[
            pl.BlockSpec((RB, H), lambda i: (i, 0)),
            pl.BlockSpec((RB,), lambda i: (i,)),
        ],
        out_specs=[
            pl.BlockSpec((G, H), lambda i: (0, 0)),
            pl.BlockSpec((G,), lambda i: (0,)),
            pl.BlockSpec((G, H), lambda i: (0, 0)),
        ],
        out_shape=[
            jax.ShapeDtypeStruct((G, H), jnp.float32),
            jax.ShapeDtypeStruct((G,), jnp.float32),
            jax.ShapeDtypeStruct((G, H), jnp.float32),
        ],
    )(comp_pad, batch_pad)


def _mlp_body(sum_ref, cnt_ref, max_ref, c1w_ref, c1b_ref, c2w_ref, c2b_ref,
              c3w_ref, c3b_ref, o_ref):
    cnt = cnt_ref[...]
    mean_pool = sum_ref[...] / jnp.maximum(cnt, 1.0)[:, None]
    max_pool = jnp.where(cnt[:, None] > 0, max_ref[...], 0.0)
    g = jnp.concatenate([mean_pool, max_pool], axis=1)
    h = jnp.dot(g, c1w_ref[...], preferred_element_type=jnp.float32)
    h = jnp.maximum(h + c1b_ref[...][None, :], 0.0)
    h = jnp.dot(h, c2w_ref[...], preferred_element_type=jnp.float32)
    h = jnp.maximum(h + c2b_ref[...][None, :], 0.0)
    o = jnp.dot(h, c3w_ref[...], preferred_element_type=jnp.float32)
    o_ref[...] = o + c3b_ref[...][None, :]


def _mlp(psum, pcnt, pmax, c1w, c1b, c2w, c2b, c3w, c3b):
    return pl.pallas_call(
        _mlp_body,
        out_shape=jax.ShapeDtypeStruct((G, 10), jnp.float32),
    )(psum, pcnt, pmax, c1w, c1b, c2w, c2b, c3w, c3b)


def _pad_rows(x, n_pad):
    n = x.shape[0]
    if n == n_pad:
        return x
    return jnp.pad(x, ((0, n_pad - n),) + ((0, 0),) * (x.ndim - 1))


def _prep_edges(ei, n_dst, n_pad):
    src, dst = ei[0].astype(jnp.int32), ei[1].astype(jnp.int32)
    e = src.shape[0]
    order = jnp.argsort(dst)
    srcs = src[order]
    dsts = dst[order]
    e_pad = _cdiv(e, EB) * EB
    srcs = jnp.pad(srcs, (0, e_pad - e))
    dsts = jnp.pad(dsts, (0, e_pad - e), constant_values=n_pad)
    q = (jnp.arange(NW + 1) * e) // NW
    cand = dsts[jnp.minimum(q, e - 1)]
    r_b = ((cand + ZR - 1) // ZR) * ZR
    r_b = jnp.minimum(r_b, n_pad)
    r_b = r_b.at[0].set(0).at[NW].set(n_pad)
    r_b = lax.cummax(r_b)
    e_b = jnp.searchsorted(dsts, r_b, side='left').astype(jnp.int32)
    pairs = jnp.zeros((NW, 16), jnp.int32)
    pairs = pairs.at[:, 0].set(e_b[:NW])
    pairs = pairs.at[:, 1].set(e_b[1:])
    pairs = pairs.at[:, 2].set(r_b[:NW])
    pairs = pairs.at[:, 3].set(r_b[1:])
    bounds = jnp.arange(n_pad + 1, dtype=jnp.int32)
    cnt = jnp.diff(jnp.searchsorted(dsts, bounds, side='left')
                   ).astype(jnp.float32)
    return srcs, dsts, pairs, e_pad, cnt


def kernel(x_component, x_pin, x_subcircuit, x_net, ei_cp, ei_pc, ei_sp,
           ei_ps, ei_pn, ei_np, batch, nte, cte, pte, Wl, bl, Wr,
           C1w, C1b, C2w, C2b, C3w, C3b):
    nc, np_, ns, nn = (x_component.shape[0], x_pin.shape[0],
                       x_subcircuit.shape[0], x_net.shape[0])
    nc_p, np_p = _cdiv(nc, RB) * RB, _cdiv(np_, RB) * RB
    ns_p, nn_p = _cdiv(ns, RB) * RB, _cdiv(nn, RB) * RB

    ecp = _prep_edges(ei_cp, np_, np_p)
    esp = _prep_edges(ei_sp, np_, np_p)
    enp = _prep_edges(ei_np, np_, np_p)
    epc = _prep_edges(ei_pc, nc, nc_p)
    eps = _prep_edges(ei_ps, ns, ns_p)
    epn = _prep_edges(ei_pn, nn, nn_p)

    zeros_h = jnp.zeros((ZR, H), jnp.float32)

    nte_p = _pad_rows(nte, 8)
    cte_p = _pad_rows(cte, 16)
    pte_p = _pad_rows(pte, 16)

    def ints3(x, n_pad, is_comp):
        x = x.astype(jnp.int32)
        nt = _pad_rows(x[:, 0], n_pad)
        if is_comp:
            ct = jnp.zeros((n_pad,), jnp.int32)
        else:
            ct = _pad_rows(jnp.clip(x[:, 1], 0), n_pad)
        pt = _pad_rows(jnp.clip(x[:, 2], 0), n_pad)
        return nt, ct, pt

    comp = _embed(*ints3(x_component, nc_p, True), nte_p, cte_p, pte_p)
    pin = _embed(*ints3(x_pin, np_p, False), nte_p, cte_p, pte_p)
    sub = _embed(*ints3(x_subcircuit, ns_p, False), nte_p, cte_p, pte_p)
    net = _embed(*ints3(x_net, nn_p, False), nte_p, cte_p, pte_p)

    c_cp, c_sp, c_np = ecp[4], esp[4], enp[4]
    c_pc, c_ps, c_pn = epc[4], eps[4], epn[4]

    def seg(prep, x_src, n_pad):
        srcs, dsts, pairs, e_pad, _ = prep
        sk = _make_seg_sum(e_pad, n_pad)
        return sk(x_src, srcs, dsts, pairs, zeros_h)

    def rows(seg_out, n_pad):
        return seg_out[:n_pad]

    for i in range(3):
        y_cp = _matmul(comp, Wl[i, 0])
        y_sp = _matmul(sub, Wl[i, 2])
        s_cp = seg(ecp, y_cp, np_p)
        s_sp = seg(esp, y_sp, np_p)
        s_np = seg(enp, net, np_p)
        s_pc = seg(epc, pin, nc_p)
        s_ps = seg(eps, pin, ns_p)
        s_pn = seg(epn, pin, nn_p)
        wr_sum = Wr[i, 0] + Wr[i, 2] + Wr[i, 5]
        b_sum = bl[i, 0] + bl[i, 2] + bl[i, 5]
        pin_new = _combine_pin(
            rows(s_cp, np_p), c_cp,
            rows(s_sp, np_p), c_sp,
            rows(s_np, np_p), c_np,
            pin, Wl[i, 5], wr_sum, b_sum)
        comp_new = _combine1(rows(s_pc, nc_p), c_pc, comp,
                             Wl[i, 1], Wr[i, 1], bl[i, 1])
        sub_new = _combine1(rows(s_ps, ns_p), c_ps, sub,
                            Wl[i, 3], Wr[i, 3], bl[i, 3])
        net_new = _combine1(rows(s_pn, nn_p), c_pn, net,
                            Wl[i, 4], Wr[i, 4], bl[i, 4])
        comp, pin, sub, net = comp_new, pin_new, sub_new, net_new

    batch_p = jnp.pad(batch.astype(jnp.int32), (0, nc_p - nc),
                      constant_values=G)
    psum, pcnt, pmax = _pool(comp, batch_p)
    return _mlp(psum, pcnt, pmax, C1w, C1b, C2w, C2b, C3w, C3b)


# speedup vs baseline: 2.6167x; 2.6167x over previous
import functools
import jax
import jax.numpy as jnp
from jax import lax
from jax.experimental import pallas as pl
from jax.experimental.pallas import tpu as pltpu
from jax.experimental.pallas import tpu_sc as plsc

H = 256
G = 64
NUM_SC = 2
NUM_SUB = 16
EB = 256
D = 4096
DPAD = 16
RB = 512
SB = 3
ZR = 64


def _cdiv(a, b):
    return (a + b - 1) // b


NW = NUM_SC * NUM_SUB


@functools.lru_cache(maxsize=None)
def _make_seg_sum(e_pad, n_out):
    mesh = plsc.VectorSubcoreMesh(
        core_axis_name="c", subcore_axis_name="s",
        num_cores=NUM_SC, num_subcores=NUM_SUB)

    @functools.partial(
        pl.kernel,
        out_type=jax.ShapeDtypeStruct((n_out + DPAD, H), jnp.float32),
        mesh=mesh,
        scratch_types=[
            pltpu.VMEM((EB,), jnp.int32),
            pltpu.VMEM((EB,), jnp.int32),
            pltpu.VMEM((EB,), jnp.int32),
            pltpu.VMEM((EB, H), jnp.float32),
            pltpu.VMEM((ZR, H), jnp.float32),
            pltpu.VMEM((1, 16), jnp.int32),
            pltpu.SemaphoreType.DMA,
            pltpu.SemaphoreType.DMA,
        ],
    )
    def kern(x_hbm, srcs_hbm, dsts_hbm, pairs_hbm, zeros_hbm, out_hbm,
             sidx, didx, lidx, rows, ztile, pairref, gsem, ssem):
        cid = lax.axis_index("c")
        sid = lax.axis_index("s")
        wid = sid * NUM_SC + cid
        pltpu.sync_copy(zeros_hbm, ztile)
        pltpu.sync_copy(pairs_hbm.at[pl.ds(wid, 1)], pairref)
        p = pairref[0, pl.ds(0, 16)]
        e_lo, e_hi, r_lo, r_hi = p[0], p[1], p[2], p[3]

        def zero_body(i, _):
            r0 = pl.multiple_of(r_lo + i * ZR, ZR)
            pltpu.sync_copy(ztile, out_hbm.at[pl.ds(r0, ZR)])
            return 0
        lax.fori_loop(0, (r_hi - r_lo) // ZR, zero_body, 0)

        blo = e_lo // EB
        bhi = (e_hi + EB - 1) // EB

        def blk_body(i, _):
            e0 = pl.multiple_of((blo + i) * EB, EB)
            pltpu.sync_copy(srcs_hbm.at[pl.ds(e0, EB)], sidx)
            pltpu.sync_copy(dsts_hbm.at[pl.ds(e0, EB)], didx)
            for k in range(EB // 16):
                v = didx[pl.ds(k * 16, 16)]
                oob = (v < r_lo) | (v >= r_hi)
                lidx[pl.ds(k * 16, 16)] = jnp.where(oob, n_out, v)
            pltpu.async_copy(x_hbm.at[sidx], rows, gsem).wait()
            pltpu.async_copy(rows, out_hbm.at[lidx], ssem, add=True).wait()
            return 0
        lax.fori_loop(0, bhi - blo, blk_body, 0)

    return kern


def _embed_body(nt_ref, ct_ref, pt_ref, nte_ref, cte_ref, pte_ref, o_ref):
    blk = nt_ref.shape[0]
    def onehot(idx, w):
        i = lax.broadcasted_iota(jnp.int32, (blk, w), 1)
        return (idx[:, None] == i).astype(jnp.float32)
    o = jnp.dot(onehot(nt_ref[...], 8), nte_ref[...],
                preferred_element_type=jnp.float32)
    o += jnp.dot(onehot(ct_ref[...], 16), cte_ref[...],
                 preferred_element_type=jnp.float32)
    o += jnp.dot(onehot(pt_ref[...], 16), pte_ref[...],
                 preferred_element_type=jnp.float32)
    o_ref[...] = o


def _embed(nt, ct, pt, nte_p, cte_p, pte_p):
    n = nt.shape[0]
    grid = (n // RB,)
    return pl.pallas_call(
        _embed_body,
        grid=grid,
        in_specs=[
            pl.BlockSpec((RB,), lambda i: (i,)),
            pl.BlockSpec((RB,), lambda i: (i,)),
            pl.BlockSpec((RB,), lambda i: (i,)),
            pl.BlockSpec((8, H), lambda i: (0, 0)),
            pl.BlockSpec((16, H), lambda i: (0, 0)),
            pl.BlockSpec((16, H), lambda i: (0, 0)),
        ],
        out_specs=pl.BlockSpec((RB, H), lambda i: (i, 0)),
        out_shape=jax.ShapeDtypeStruct((n, H), jnp.float32),
    )(nt, ct, pt, nte_p, cte_p, pte_p)


def _matmul_body(x_ref, w_ref, o_ref):
    o_ref[...] = jnp.dot(x_ref[...], w_ref[...],
                         preferred_element_type=jnp.float32)


def _matmul(x, w):
    n = x.shape[0]
    return pl.pallas_call(
        _matmul_body,
        grid=(n // RB,),
        in_specs=[
            pl.BlockSpec((RB, H), lambda i: (i, 0)),
            pl.BlockSpec((H, H), lambda i: (0, 0)),
        ],
        out_specs=pl.BlockSpec((RB, H), lambda i: (i, 0)),
        out_shape=jax.ShapeDtypeStruct((n, H), jnp.float32),
    )(x, w)


def _combine1_body(s_ref, c_ref, x_ref, wl_ref, wr_ref, b_ref, o_ref):
    a = s_ref[...] / jnp.maximum(c_ref[...], 1.0)[:, None]
    o = jnp.dot(a, wl_ref[...], preferred_element_type=jnp.float32)
    o += jnp.dot(x_ref[...], wr_ref[...], preferred_element_type=jnp.float32)
    o_ref[...] = jnp.maximum(o + b_ref[...][None, :], 0.0)


def _combine1(s, c, x, wl, wr, b):
    n = x.shape[0]
    return pl.pallas_call(
        _combine1_body,
        grid=(n // RB,),
        in_specs=[
            pl.BlockSpec((RB, H), lambda i: (i, 0)),
            pl.BlockSpec((RB,), lambda i: (i,)),
            pl.BlockSpec((RB, H), lambda i: (i, 0)),
            pl.BlockSpec((H, H), lambda i: (0, 0)),
            pl.BlockSpec((H, H), lambda i: (0, 0)),
            pl.BlockSpec((H,), lambda i: (0,)),
        ],
        out_specs=pl.BlockSpec((RB, H), lambda i: (i, 0)),
        out_shape=jax.ShapeDtypeStruct((n, H), jnp.float32),
    )(s, c, x, wl, wr, b)


def _combine_pin_body(s1_ref, c1_ref, s2_ref, c2_ref, s3_ref, c3_ref,
                      x_ref, wl_ref, wr_ref, b_ref, o_ref):
    o = s1_ref[...] / jnp.maximum(c1_ref[...], 1.0)[:, None]
    o += s2_ref[...] / jnp.maximum(c2_ref[...], 1.0)[:, None]
    a3 = s3_ref[...] / jnp.maximum(c3_ref[...], 1.0)[:, None]
    o += jnp.dot(a3, wl_ref[...], preferred_element_type=jnp.float32)
    o += jnp.dot(x_ref[...], wr_ref[...], preferred_element_type=jnp.float32)
    o_ref[...] = jnp.maximum(o + b_ref[...][None, :], 0.0)


def _combine_pin(s1, c1, s2, c2, s3, c3, x, wl3, wr, b):
    n = x.shape[0]
    row = pl.BlockSpec((RB, H), lambda i: (i, 0))
    vec = pl.BlockSpec((RB,), lambda i: (i,))
    return pl.pallas_call(
        _combine_pin_body,
        grid=(n // RB,),
        in_specs=[row, vec, row, vec, row, vec, row,
                  pl.BlockSpec((H, H), lambda i: (0, 0)),
                  pl.BlockSpec((H, H), lambda i: (0, 0)),
                  pl.BlockSpec((H,), lambda i: (0,))],
        out_specs=row,
        out_shape=jax.ShapeDtypeStruct((n, H), jnp.float32),
    )(s1, c1, s2, c2, s3, c3, x, wl3, wr, b)


def _pool_body(comp_ref, batch_ref, sum_ref, cnt_ref, max_ref):
    i = pl.program_id(0)
    blk = comp_ref.shape[0]

    @pl.when(i == 0)
    def _init():
        sum_ref[...] = jnp.zeros_like(sum_ref)
        cnt_ref[...] = jnp.zeros_like(cnt_ref)
        max_ref[...] = jnp.full_like(max_ref, -3.0e38)

    comp = comp_ref[...]
    b = batch_ref[...]
    oh = (b[:, None] == lax.broadcasted_iota(jnp.int32, (blk, G), 1))
    ohf = oh.astype(jnp.float32)
    sum_ref[...] += jnp.dot(ohf.T, comp, preferred_element_type=jnp.float32)
    cnt_ref[...] += jnp.sum(ohf, axis=0)
    g0 = b[0]
    g1 = b[blk - 1]
    for g in range(G):
        @pl.when((g0 <= g) & (g <= g1))
        def _gmax():
            m = jnp.where(oh[:, g][:, None], comp, -3.0e38)
            max_ref[g, :] = jnp.maximum(max_ref[g, :], jnp.max(m, axis=0))


def _pool(comp_pad, batch_pad):
    n = comp_pad.shape[0]
    return pl.pallas_call(
        _pool_body,
        grid=(n // RB,),
        in_specs=[
            pl.BlockSpec((RB, H), lambda i: (i, 0)),
            pl.BlockSpec((RB,), lambda i: (i,)),
        ],
        out_specs=[
            pl.BlockSpec((G, H), lambda i: (0, 0)),
            pl.BlockSpec((G,), lambda i: (0,)),
            pl.BlockSpec((G, H), lambda i: (0, 0)),
        ],
        out_shape=[
            jax.ShapeDtypeStruct((G, H), jnp.float32),
            jax.ShapeDtypeStruct((G,), jnp.float32),
            jax.ShapeDtypeStruct((G, H), jnp.float32),
        ],
    )(comp_pad, batch_pad)


def _mlp_body(sum_ref, cnt_ref, max_ref, c1w_ref, c1b_ref, c2w_ref, c2b_ref,
              c3w_ref, c3b_ref, o_ref):
    cnt = cnt_ref[...]
    mean_pool = sum_ref[...] / jnp.maximum(cnt, 1.0)[:, None]
    max_pool = jnp.where(cnt[:, None] > 0, max_ref[...], 0.0)
    g = jnp.concatenate([mean_pool, max_pool], axis=1)
    h = jnp.dot(g, c1w_ref[...], preferred_element_type=jnp.float32)
    h = jnp.maximum(h + c1b_ref[...][None, :], 0.0)
    h = jnp.dot(h, c2w_ref[...], preferred_element_type=jnp.float32)
    h = jnp.maximum(h + c2b_ref[...][None, :], 0.0)
    o = jnp.dot(h, c3w_ref[...], preferred_element_type=jnp.float32)
    o_ref[...] = o + c3b_ref[...][None, :]


def _mlp(psum, pcnt, pmax, c1w, c1b, c2w, c2b, c3w, c3b):
    return pl.pallas_call(
        _mlp_body,
        out_shape=jax.ShapeDtypeStruct((G, 10), jnp.float32),
    )(psum, pcnt, pmax, c1w, c1b, c2w, c2b, c3w, c3b)


def _pad_rows(x, n_pad):
    n = x.shape[0]
    if n == n_pad:
        return x
    return jnp.pad(x, ((0, n_pad - n),) + ((0, 0),) * (x.ndim - 1))


def _prep_edges(ei, n_dst, n_pad):
    src, dst = ei[0].astype(jnp.int32), ei[1].astype(jnp.int32)
    e = src.shape[0]
    order = jnp.argsort(dst)
    srcs = src[order]
    dsts = dst[order]
    e_pad = _cdiv(e, EB) * EB
    srcs = jnp.pad(srcs, (0, e_pad - e))
    dsts = jnp.pad(dsts, (0, e_pad - e), constant_values=n_pad)
    q = (jnp.arange(NW + 1) * e) // NW
    cand = dsts[jnp.minimum(q, e - 1)]
    r_b = ((cand + ZR - 1) // ZR) * ZR
    r_b = jnp.minimum(r_b, n_pad)
    r_b = r_b.at[0].set(0).at[NW].set(n_pad)
    r_b = lax.cummax(r_b)
    e_b = jnp.searchsorted(dsts, r_b, side='left').astype(jnp.int32)
    pairs = jnp.zeros((NW, 16), jnp.int32)
    pairs = pairs.at[:, 0].set(e_b[:NW])
    pairs = pairs.at[:, 1].set(e_b[1:])
    pairs = pairs.at[:, 2].set(r_b[:NW])
    pairs = pairs.at[:, 3].set(r_b[1:])
    cnt = jnp.zeros((n_pad,), jnp.float32).at[dst].add(1.0)
    return srcs, dsts, pairs, e_pad, cnt


def kernel(x_component, x_pin, x_subcircuit, x_net, ei_cp, ei_pc, ei_sp,
           ei_ps, ei_pn, ei_np, batch, nte, cte, pte, Wl, bl, Wr,
           C1w, C1b, C2w, C2b, C3w, C3b):
    nc, np_, ns, nn = (x_component.shape[0], x_pin.shape[0],
                       x_subcircuit.shape[0], x_net.shape[0])
    nc_p, np_p = _cdiv(nc, RB) * RB, _cdiv(np_, RB) * RB
    ns_p, nn_p = _cdiv(ns, RB) * RB, _cdiv(nn, RB) * RB

    ecp = _prep_edges(ei_cp, np_, np_p)
    esp = _prep_edges(ei_sp, np_, np_p)
    enp = _prep_edges(ei_np, np_, np_p)
    epc = _prep_edges(ei_pc, nc, nc_p)
    eps = _prep_edges(ei_ps, ns, ns_p)
    epn = _prep_edges(ei_pn, nn, nn_p)

    zeros_h = jnp.zeros((ZR, H), jnp.float32)

    nte_p = _pad_rows(nte, 8)
    cte_p = _pad_rows(cte, 16)
    pte_p = _pad_rows(pte, 16)

    def ints3(x, n_pad, is_comp):
        x = x.astype(jnp.int32)
        nt = _pad_rows(x[:, 0], n_pad)
        if is_comp:
            ct = jnp.zeros((n_pad,), jnp.int32)
        else:
            ct = _pad_rows(jnp.clip(x[:, 1], 0), n_pad)
        pt = _pad_rows(jnp.clip(x[:, 2], 0), n_pad)
        return nt, ct, pt

    comp = _embed(*ints3(x_component, nc_p, True), nte_p, cte_p, pte_p)
    pin = _embed(*ints3(x_pin, np_p, False), nte_p, cte_p, pte_p)
    sub = _embed(*ints3(x_subcircuit, ns_p, False), nte_p, cte_p, pte_p)
    net = _embed(*ints3(x_net, nn_p, False), nte_p, cte_p, pte_p)

    c_cp, c_sp, c_np = ecp[4], esp[4], enp[4]
    c_pc, c_ps, c_pn = epc[4], eps[4], epn[4]

    def seg(prep, x_src, n_pad):
        srcs, dsts, pairs, e_pad, _ = prep
        sk = _make_seg_sum(e_pad, n_pad)
        return sk(x_src, srcs, dsts, pairs, zeros_h)

    def rows(seg_out, n_pad):
        return seg_out[:n_pad]

    for i in range(3):
        y_cp = _matmul(comp, Wl[i, 0])
        y_sp = _matmul(sub, Wl[i, 2])
        s_cp = seg(ecp, y_cp, np_p)
        s_sp = seg(esp, y_sp, np_p)
        s_np = seg(enp, net, np_p)
        s_pc = seg(epc, pin, nc_p)
        s_ps = seg(eps, pin, ns_p)
        s_pn = seg(epn, pin, nn_p)
        wr_sum = Wr[i, 0] + Wr[i, 2] + Wr[i, 5]
        b_sum = bl[i, 0] + bl[i, 2] + bl[i, 5]
        pin_new = _combine_pin(
            rows(s_cp, np_p), c_cp,
            rows(s_sp, np_p), c_sp,
            rows(s_np, np_p), c_np,
            pin, Wl[i, 5], wr_sum, b_sum)
        comp_new = _combine1(rows(s_pc, nc_p), c_pc, comp,
                             Wl[i, 1], Wr[i, 1], bl[i, 1])
        sub_new = _combine1(rows(s_ps, ns_p), c_ps, sub,
                            Wl[i, 3], Wr[i, 3], bl[i, 3])
        net_new = _combine1(rows(s_pn, nn_p), c_pn, net,
                            Wl[i, 4], Wr[i, 4], bl[i, 4])
        comp, pin, sub, net = comp_new, pin_new, sub_new, net_new

    batch_p = jnp.pad(batch.astype(jnp.int32), (0, nc_p - nc),
                      constant_values=G)
    psum, pcnt, pmax = _pool(comp, batch_p)
    return _mlp(psum, pcnt, pmax, C1w, C1b, C2w, C2b, C3w, C3b)
